# baseline (speedup 1.0000x reference)
import math

import jax
import jax.numpy as jnp
from jax import lax
from jax.experimental import pallas as pl
from jax.experimental.pallas import tpu as pltpu
from jax.experimental.pallas import tpu_sc as plsc

_SMOOTH = 0.1
_V = 32000
_B = 4096
_OFF = _SMOOTH / (_V - 2)
_ON = 1.0 - _SMOOTH
_C0 = _SMOOTH * math.log(_OFF) + _ON * math.log(_ON)

_CS = 17920
_CW = _V - _CS

_NC = 2
_NS = 16
_L = 16
_NW = _NC * _NS
_RPW = _B // _NW

_BR = 256
_BC = 1280
_NRB = _B // _BR
_NCB = _CS // _BC


def _sc_body(x2d_hbm, tgt_hbm, out_hbm, tgt_v, rv_v, part_v,
             b0, b1, b2, b3, s0, s1, s2, s3):
    wid = lax.axis_index("s") * _NC + lax.axis_index("c")
    base = wid * _RPW
    iota16 = lax.iota(jnp.int32, _L)
    ring = ((b0, s0), (b1, s1), (b2, s2), (b3, s3))
    _NB = len(ring)

    pltpu.sync_copy(tgt_hbm.at[pl.ds(base, _RPW)], tgt_v)

    for k, (bf, sm) in enumerate(ring):
        pltpu.make_async_copy(x2d_hbm.at[base + k, pl.ds(_CS, _CW)], bf, sm).start()

    def row_sum(bf):
        def it_body(it, accs):
            a0, a1, a2, a3 = accs
            o = it * 128
            a0 = a0 + bf[pl.ds(o, _L)] + bf[pl.ds(o + 64, _L)]
            a1 = a1 + bf[pl.ds(o + 16, _L)] + bf[pl.ds(o + 80, _L)]
            a2 = a2 + bf[pl.ds(o + 32, _L)] + bf[pl.ds(o + 96, _L)]
            a3 = a3 + bf[pl.ds(o + 48, _L)] + bf[pl.ds(o + 112, _L)]
            return a0, a1, a2, a3
        z = jnp.zeros((_L,), jnp.float32)
        a0, a1, a2, a3 = lax.fori_loop(0, _CW // 128, it_body, (z, z, z, z))
        return jnp.sum((a0 + a1) + (a2 + a3))

    def group(p, gacc16):
        for k, (bf, sm) in enumerate(ring):
            r = _NB * p + k
            pltpu.make_async_copy(x2d_hbm.at[0, pl.ds(_CS, _CW)], bf, sm).wait()
            s = row_sum(bf)
            rl = r & 15
            b16 = r - rl
            tgt16 = tgt_v[pl.ds(b16, _L)]
            t_r = jnp.sum(jnp.where(iota16 == rl, tgt16, 0))
            inb = t_r >= _CS
            idx16 = iota16 * 0 + jnp.where(inb, t_r - _CS, 0)
            gat16 = plsc.load_gather(bf, [idx16])
            gacc16 = gacc16 + jnp.where((iota16 == 0) & inb, gat16, 0.0)
            old = rv_v[pl.ds(b16, _L)]
            rv_v[pl.ds(b16, _L)] = jnp.where(iota16 == rl, s, old)

            @pl.when(r + _NB < _RPW)
            def _next():
                pltpu.make_async_copy(
                    x2d_hbm.at[base + r + _NB, pl.ds(_CS, _CW)], bf, sm
                ).start()
        return gacc16

    gacc16 = lax.fori_loop(
        0, _RPW // _NB, group, jnp.zeros((_L,), jnp.float32)
    )

    total16 = (_OFF - _ON) * gacc16
    for g in range(_RPW // _L):
        t16 = tgt_v[pl.ds(g * _L, _L)]
        valid = t16 != 0
        rv16 = rv_v[pl.ds(g * _L, _L)]
        total16 = total16 + jnp.where(valid, -_OFF * rv16, 0.0)
    part_v[...] = total16
    pltpu.sync_copy(part_v, out_hbm.at[wid])


def _sc_loss(x, target):
    mesh = plsc.VectorSubcoreMesh(
        core_axis_name="c", subcore_axis_name="s", num_cores=_NC, num_subcores=_NS
    )
    return pl.kernel(
        _sc_body,
        out_type=jax.ShapeDtypeStruct((_NW, _L), jnp.float32),
        mesh=mesh,
        compiler_params=pltpu.CompilerParams(needs_layout_passes=False),
        scratch_types=[
            pltpu.VMEM((_RPW,), jnp.int32),
            pltpu.VMEM((_RPW,), jnp.float32),
            pltpu.VMEM((_L,), jnp.float32),
            pltpu.VMEM((_CW,), jnp.float32),
            pltpu.VMEM((_CW,), jnp.float32),
            pltpu.VMEM((_CW,), jnp.float32),
            pltpu.VMEM((_CW,), jnp.float32),
            pltpu.SemaphoreType.DMA,
            pltpu.SemaphoreType.DMA,
            pltpu.SemaphoreType.DMA,
            pltpu.SemaphoreType.DMA,
        ],
    )(x, target)


def _tc_body(x_ref, tgt_ref, out_ref):
    i = pl.program_id(0)
    j = pl.program_id(1)

    @pl.when((i == 0) & (j == 0))
    def _init():
        out_ref[0, 0] = 0.0

    tgt2 = tgt_ref[i, :, :]
    valid2 = tgt2 != 0
    xb = x_ref[...]
    rs2 = jnp.sum(xb, axis=1, keepdims=True)
    acc = -_OFF * jnp.sum(jnp.where(valid2, rs2, 0.0))

    cols = j * _BC + lax.broadcasted_iota(jnp.int32, (_BR, _BC), 1)
    match = (tgt2 == cols) & valid2
    acc += (_OFF - _ON) * jnp.sum(jnp.where(match, xb, 0.0))

    @pl.when(j == 0)
    def _col0_and_const():
        col0 = xb[:, 0:1]
        out_ref[0, 0] += _OFF * jnp.sum(
            jnp.where(valid2, col0, 0.0)
        ) + _C0 * jnp.sum(jnp.where(valid2, 1.0, 0.0))

    out_ref[0, 0] += acc


def _tc_reduce(x, tgt3d):
    return pl.pallas_call(
        _tc_body,
        grid=(_NRB, _NCB),
        in_specs=[
            pl.BlockSpec((_BR, _BC), lambda i, j: (i, j)),
            pl.BlockSpec((_NRB, _BR, 1), lambda i, j: (0, 0, 0)),
        ],
        out_specs=pl.BlockSpec((1, 1), lambda i, j: (0, 0), memory_space=pltpu.SMEM),
        out_shape=jax.ShapeDtypeStruct((1, 1), jnp.float32),
        compiler_params=pltpu.CompilerParams(
            dimension_semantics=("arbitrary", "arbitrary")
        ),
    )(x, tgt3d)


@jax.jit
def kernel(x, target):
    target = target.astype(jnp.int32)
    sc_parts = _sc_loss(x, target)
    tgt3d = jnp.reshape(target, (_NRB, _BR, 1))
    out = _tc_reduce(x, tgt3d)
    return out[0, 0] + jnp.sum(sc_parts)


# speedup vs baseline: 1.3940x; 1.3940x over previous
import math

import jax
import jax.numpy as jnp
from jax import lax
from jax.experimental import pallas as pl
from jax.experimental.pallas import tpu as pltpu
from jax.experimental.pallas import tpu_sc as plsc

_SMOOTH = 0.1
_V = 32000
_B = 4096
_OFF = _SMOOTH / (_V - 2)
_ON = 1.0 - _SMOOTH
_C0 = _SMOOTH * math.log(_OFF) + _ON * math.log(_ON)

_CS = 16000
_CW = _V - _CS

_NC = 2
_NS = 16
_L = 16
_NW = _NC * _NS
_RPW = _B // _NW

_BR = 256
_BC = 3200
_NRB = _B // _BR
_NCB = _CS // _BC


def _sc_body(x2d_hbm, tgt_hbm, out_hbm, tgt_v, rv_v, part_v,
             b0, b1, b2, b3, s0, s1, s2, s3):
    wid = lax.axis_index("s") * _NC + lax.axis_index("c")
    base = wid * _RPW
    iota16 = lax.iota(jnp.int32, _L)
    ring = ((b0, s0), (b1, s1), (b2, s2), (b3, s3))
    _NB = len(ring)

    pltpu.sync_copy(tgt_hbm.at[pl.ds(base, _RPW)], tgt_v)

    for k, (bf, sm) in enumerate(ring):
        pltpu.make_async_copy(x2d_hbm.at[base + k, pl.ds(_CS, _CW)], bf, sm).start()

    def row_sum(bf):
        def it_body(it, accs):
            a0, a1, a2, a3 = accs
            o = it * 128
            a0 = a0 + bf[pl.ds(o, _L)] + bf[pl.ds(o + 64, _L)]
            a1 = a1 + bf[pl.ds(o + 16, _L)] + bf[pl.ds(o + 80, _L)]
            a2 = a2 + bf[pl.ds(o + 32, _L)] + bf[pl.ds(o + 96, _L)]
            a3 = a3 + bf[pl.ds(o + 48, _L)] + bf[pl.ds(o + 112, _L)]
            return a0, a1, a2, a3
        z = jnp.zeros((_L,), jnp.float32)
        a0, a1, a2, a3 = lax.fori_loop(0, _CW // 128, it_body, (z, z, z, z))
        return jnp.sum((a0 + a1) + (a2 + a3))

    def group(p, gacc16):
        for k, (bf, sm) in enumerate(ring):
            r = _NB * p + k
            pltpu.make_async_copy(x2d_hbm.at[0, pl.ds(_CS, _CW)], bf, sm).wait()
            s = row_sum(bf)
            rl = r & 15
            b16 = r - rl
            tgt16 = tgt_v[pl.ds(b16, _L)]
            t_r = jnp.sum(jnp.where(iota16 == rl, tgt16, 0))
            inb = t_r >= _CS
            idx16 = iota16 * 0 + jnp.where(inb, t_r - _CS, 0)
            gat16 = plsc.load_gather(bf, [idx16])
            gacc16 = gacc16 + jnp.where((iota16 == 0) & inb, gat16, 0.0)
            old = rv_v[pl.ds(b16, _L)]
            rv_v[pl.ds(b16, _L)] = jnp.where(iota16 == rl, s, old)

            @pl.when(r + _NB < _RPW)
            def _next():
                pltpu.make_async_copy(
                    x2d_hbm.at[base + r + _NB, pl.ds(_CS, _CW)], bf, sm
                ).start()
        return gacc16

    gacc16 = lax.fori_loop(
        0, _RPW // _NB, group, jnp.zeros((_L,), jnp.float32)
    )

    total16 = (_OFF - _ON) * gacc16
    for g in range(_RPW // _L):
        t16 = tgt_v[pl.ds(g * _L, _L)]
        valid = t16 != 0
        rv16 = rv_v[pl.ds(g * _L, _L)]
        total16 = total16 + jnp.where(valid, -_OFF * rv16, 0.0)
    part_v[...] = total16
    pltpu.sync_copy(part_v, out_hbm.at[wid])


def _sc_loss(x, target):
    mesh = plsc.VectorSubcoreMesh(
        core_axis_name="c", subcore_axis_name="s", num_cores=_NC, num_subcores=_NS
    )
    return pl.kernel(
        _sc_body,
        out_type=jax.ShapeDtypeStruct((_NW, _L), jnp.float32),
        mesh=mesh,
        compiler_params=pltpu.CompilerParams(needs_layout_passes=False),
        scratch_types=[
            pltpu.VMEM((_RPW,), jnp.int32),
            pltpu.VMEM((_RPW,), jnp.float32),
            pltpu.VMEM((_L,), jnp.float32),
            pltpu.VMEM((_CW,), jnp.float32),
            pltpu.VMEM((_CW,), jnp.float32),
            pltpu.VMEM((_CW,), jnp.float32),
            pltpu.VMEM((_CW,), jnp.float32),
            pltpu.SemaphoreType.DMA,
            pltpu.SemaphoreType.DMA,
            pltpu.SemaphoreType.DMA,
            pltpu.SemaphoreType.DMA,
        ],
    )(x, target)


def _tc_body(x_ref, tgt_ref, out_ref):
    i = pl.program_id(0)
    j = pl.program_id(1)

    @pl.when((i == 0) & (j == 0))
    def _init():
        out_ref[0, 0] = 0.0

    tgt2 = tgt_ref[i, :, :]
    valid2 = tgt2 != 0
    xb = x_ref[...]
    rs2 = jnp.sum(xb, axis=1, keepdims=True)
    acc = -_OFF * jnp.sum(jnp.where(valid2, rs2, 0.0))

    cols = j * _BC + lax.broadcasted_iota(jnp.int32, (_BR, _BC), 1)
    match = (tgt2 == cols) & valid2
    acc += (_OFF - _ON) * jnp.sum(jnp.where(match, xb, 0.0))

    @pl.when(j == 0)
    def _col0_and_const():
        col0 = xb[:, 0:1]
        out_ref[0, 0] += _OFF * jnp.sum(
            jnp.where(valid2, col0, 0.0)
        ) + _C0 * jnp.sum(jnp.where(valid2, 1.0, 0.0))

    out_ref[0, 0] += acc


def _tc_reduce(x, tgt3d):
    return pl.pallas_call(
        _tc_body,
        grid=(_NRB, _NCB),
        in_specs=[
            pl.BlockSpec((_BR, _BC), lambda i, j: (i, j)),
            pl.BlockSpec((_NRB, _BR, 1), lambda i, j: (0, 0, 0)),
        ],
        out_specs=pl.BlockSpec((1, 1), lambda i, j: (0, 0), memory_space=pltpu.SMEM),
        out_shape=jax.ShapeDtypeStruct((1, 1), jnp.float32),
        compiler_params=pltpu.CompilerParams(
            dimension_semantics=("arbitrary", "arbitrary")
        ),
    )(x, tgt3d)


@jax.jit
def kernel(x, target):
    target = target.astype(jnp.int32)
    sc_parts = _sc_loss(x, target)
    tgt3d = jnp.reshape(target, (_NRB, _BR, 1))
    out = _tc_reduce(x, tgt3d)
    return out[0, 0] + jnp.sum(sc_parts)
